# baseline (speedup 1.0000x reference)
import functools

import jax
import jax.numpy as jnp
from jax import lax
from jax.experimental import pallas as pl
from jax.experimental.pallas import tpu as pltpu
from jax.experimental.pallas import tpu_sc as plsc

_NC = 2
_NS = 16
_NW = _NC * _NS


def _tc_repack(embed_t, v4p):
    D, V = embed_t.shape
    cols = 4096
    nblk = v4p // cols
    last_blk = (V - 1) // cols

    def repack_body(x0, x1, x2, x3, o_ref):
        xcat = jnp.concatenate(
            [x0[...], x1[...], x2[...], x3[...]], axis=0)
        o_ref[...] = xcat.T

    in_specs = [
        pl.BlockSpec(
            (D, cols),
            functools.partial(
                lambda j, i: (0, jnp.minimum(i + j * nblk, last_blk)), j))
        for j in range(4)
    ]
    return pl.pallas_call(
        repack_body,
        grid=(nblk,),
        in_specs=in_specs,
        out_specs=pl.BlockSpec((cols, 4 * D), lambda i: (i, 0)),
        out_shape=jax.ShapeDtypeStruct((v4p, 4 * D), embed_t.dtype),
    )(embed_t, embed_t, embed_t, embed_t)


def _sc_permute_ids(ids_tm, n, Bc, shift):
    q4 = Bc // 4
    b_per_w = n // _NW
    chunk = 1024
    n_chunks = b_per_w // chunk
    mask = (1 << shift) - 1
    mesh = plsc.VectorSubcoreMesh(core_axis_name="c", subcore_axis_name="s")

    @functools.partial(
        pl.kernel,
        mesh=mesh,
        out_type=jax.ShapeDtypeStruct((n,), jnp.int32),
        compiler_params=pltpu.CompilerParams(
            use_tc_tiling_on_sc=False, needs_layout_passes=False),
        scratch_types=[
            pltpu.VMEM((chunk,), jnp.int32),
            pltpu.VMEM((chunk,), jnp.int32),
        ],
    )
    def permute_kernel(idx_hbm, out_hbm, src_v, idx_v):
        wid = lax.axis_index("s") * _NC + lax.axis_index("c")
        base = wid * b_per_w
        r = lax.iota(jnp.int32, 16)
        pat = (chunk // 4) * (r & 3) + (r >> 2)

        @pl.loop(0, n_chunks)
        def _(ci):
            off = base + ci * chunk
            t_base = (off // Bc) * Bc
            c = (off - t_base) // chunk
            for j in range(4):
                s = t_base + q4 * j + (chunk // 4) * c
                pltpu.sync_copy(idx_hbm.at[pl.ds(s, chunk // 4)],
                                src_v.at[pl.ds(j * (chunk // 4), chunk // 4)])

            @pl.loop(0, chunk // 16)
            def _(m):
                g = plsc.load_gather(src_v, [pat + 4 * m])
                rid = 4 * (g & mask) + (g >> shift)
                idx_v[pl.ds(m * 16, 16)] = rid

            pltpu.sync_copy(idx_v, out_hbm.at[pl.ds(off, chunk)])

    return permute_kernel(ids_tm)


def _sc_gather(table, ids_f, start, cnt, chunk):
    d = table.shape[1]
    b_per_w = cnt // _NW
    n_chunks = b_per_w // chunk
    assert n_chunks % 2 == 1
    mesh = plsc.VectorSubcoreMesh(core_axis_name="c", subcore_axis_name="s")

    @functools.partial(
        pl.kernel,
        mesh=mesh,
        out_type=jax.ShapeDtypeStruct((cnt, d), table.dtype),
        compiler_params=pltpu.CompilerParams(
            use_tc_tiling_on_sc=False, needs_layout_passes=False),
        scratch_types=[
            pltpu.VMEM((chunk,), jnp.int32),
            pltpu.VMEM((chunk,), jnp.int32),
            pltpu.VMEM((chunk, d), table.dtype),
            pltpu.VMEM((chunk, d), table.dtype),
            pltpu.SemaphoreType.DMA,
            pltpu.SemaphoreType.DMA,
            pltpu.SemaphoreType.DMA,
            pltpu.SemaphoreType.DMA,
            pltpu.SemaphoreType.DMA,
            pltpu.SemaphoreType.DMA,
        ],
    )
    def gather_kernel(table_hbm, idx_hbm, out_hbm,
                      idx_v0, idx_v1, rows_v0, rows_v1,
                      si0, si1, sg0, sg1, so0, so1):
        wid = lax.axis_index("s") * _NC + lax.axis_index("c")
        base = wid * b_per_w
        idx_v = (idx_v0, idx_v1)
        rows_v = (rows_v0, rows_v1)
        si = (si0, si1)
        sg = (sg0, sg1)
        so = (so0, so1)

        def do_chunk(cc, b, prefetch, out_wait):
            off = base + cc * chunk
            if prefetch:
                @pl.when(cc + 1 < n_chunks)
                def _():
                    noff = base + (cc + 1) * chunk
                    pltpu.async_copy(idx_hbm.at[pl.ds(start + noff, chunk)],
                                     idx_v[1 - b], si[1 - b])
            pltpu.make_async_copy(idx_hbm.at[pl.ds(start + off, chunk)],
                                  idx_v[b], si[b]).wait()
            if out_wait:
                @pl.when(cc >= 2)
                def _():
                    pltpu.make_async_copy(rows_v[b],
                                          out_hbm.at[pl.ds(off, chunk)],
                                          so[b]).wait()
            pltpu.async_copy(table_hbm.at[idx_v[b]], rows_v[b], sg[b]).wait()
            pltpu.async_copy(rows_v[b], out_hbm.at[pl.ds(off, chunk)], so[b])

        pltpu.async_copy(idx_hbm.at[pl.ds(start + base, chunk)], idx_v0, si0)

        @pl.loop(0, n_chunks - 1, step=2)
        def _(ci):
            do_chunk(ci, 0, True, True)
            do_chunk(ci + 1, 1, True, True)

        do_chunk(n_chunks - 1, 0, False, True)

        pltpu.make_async_copy(
            rows_v1, out_hbm.at[pl.ds(base + (n_chunks - 2) * chunk, chunk)],
            so1).wait()
        pltpu.make_async_copy(
            rows_v0, out_hbm.at[pl.ds(base + (n_chunks - 1) * chunk, chunk)],
            so0).wait()

    return gather_kernel(table, ids_f)


def _tc_proj(x2, Wblk, bcol, Tc, Bc, D, t0, nt, prev=None):
    q = Bc // 4
    tblk = 4

    def proj_body(*refs):
        x_ref, w_ref, b_ref = refs[0], refs[1], refs[2]
        o_ref = refs[-1]
        dn = (((1,), (1,)), ((), ()))
        for tt in range(tblk):
            xt = x_ref[tt * q:(tt + 1) * q, :]
            z = lax.dot_general(w_ref[...], xt, dn,
                                preferred_element_type=jnp.float32)
            z = z + b_ref[...]
            for j in range(4):
                o_ref[tt, :, j * q:(j + 1) * q] = z[j * D:(j + 1) * D, :]

    in_specs = [
        pl.BlockSpec((tblk * q, 4 * D), lambda i: (i, 0)),
        pl.BlockSpec((4 * D, 4 * D), lambda i: (0, 0)),
        pl.BlockSpec((4 * D, 1), lambda i: (0, 0)),
    ]
    args = [x2, Wblk, bcol]
    kwargs = {}
    if prev is not None:
        in_specs.append(pl.BlockSpec(memory_space=pl.ANY))
        args.append(prev)
        kwargs["input_output_aliases"] = {3: 0}
    off = t0 // tblk
    return pl.pallas_call(
        proj_body,
        grid=(nt // tblk,),
        in_specs=in_specs,
        out_specs=pl.BlockSpec((tblk, D, Bc), lambda i: (i + off, 0, 0)),
        out_shape=jax.ShapeDtypeStruct((Tc, D, Bc), jnp.float32),
        **kwargs,
    )(*args)


def kernel(input_ids, embed, W, b):
    Bc, Tc = input_ids.shape
    V, D = embed.shape
    n = Bc * Tc
    q = Bc // 4

    shift = max(int(V - 1).bit_length() - 2, 1)
    v4p = 1 << shift

    table4 = _tc_repack(embed.T, v4p)
    table_lin = table4.reshape(4 * v4p, D)

    ids_tm = input_ids.T.reshape(n)
    ids_f = _sc_permute_ids(ids_tm, n, Bc, shift)

    Wblk = jnp.kron(jnp.eye(4, dtype=W.dtype), W)
    bcol = jnp.tile(b, 4)[:, None]
    half = n // 2
    chunk = half // _NW // 25
    x_a = _sc_gather(table_lin, ids_f, 0, half, chunk)
    x_b = _sc_gather(table_lin, ids_f, half, half, chunk)
    x2_a = x_a.reshape(half // 4, 4 * D)
    x2_b = x_b.reshape(half // 4, 4 * D)

    yt_a = _tc_proj(x2_a, Wblk, bcol, Tc, Bc, D, 0, Tc // 2)
    yt = _tc_proj(x2_b, Wblk, bcol, Tc, Bc, D, Tc // 2, Tc // 2, prev=yt_a)

    return jnp.transpose(yt, (2, 0, 1))


# speedup vs baseline: 1.0153x; 1.0153x over previous
import functools

import jax
import jax.numpy as jnp
from jax import lax
from jax.experimental import pallas as pl
from jax.experimental.pallas import tpu as pltpu
from jax.experimental.pallas import tpu_sc as plsc

_NC = 2
_NS = 16
_NW = _NC * _NS


def _tc_repack(embed_t, v4p):
    D, V = embed_t.shape
    cols = 4096
    nblk = v4p // cols
    last_blk = (V - 1) // cols

    def repack_body(x0, x1, x2, x3, o_ref):
        xcat = jnp.concatenate(
            [x0[...], x1[...], x2[...], x3[...]], axis=0)
        o_ref[...] = xcat.T

    in_specs = [
        pl.BlockSpec(
            (D, cols),
            functools.partial(
                lambda j, i: (0, jnp.minimum(i + j * nblk, last_blk)), j))
        for j in range(4)
    ]
    return pl.pallas_call(
        repack_body,
        grid=(nblk,),
        in_specs=in_specs,
        out_specs=pl.BlockSpec((cols, 4 * D), lambda i: (i, 0)),
        out_shape=jax.ShapeDtypeStruct((v4p, 4 * D), embed_t.dtype),
    )(embed_t, embed_t, embed_t, embed_t)


def _sc_permute_ids(ids_tm, n, Bc, shift):
    q4 = Bc // 4
    b_per_w = n // _NW
    chunk = 1024
    n_chunks = b_per_w // chunk
    mask = (1 << shift) - 1
    mesh = plsc.VectorSubcoreMesh(core_axis_name="c", subcore_axis_name="s")

    @functools.partial(
        pl.kernel,
        mesh=mesh,
        out_type=jax.ShapeDtypeStruct((n,), jnp.int32),
        compiler_params=pltpu.CompilerParams(
            use_tc_tiling_on_sc=False, needs_layout_passes=False),
        scratch_types=[
            pltpu.VMEM((chunk,), jnp.int32),
            pltpu.VMEM((chunk,), jnp.int32),
        ],
    )
    def permute_kernel(idx_hbm, out_hbm, src_v, idx_v):
        wid = lax.axis_index("s") * _NC + lax.axis_index("c")
        base = wid * b_per_w
        r = lax.iota(jnp.int32, 16)
        pat = (chunk // 4) * (r & 3) + (r >> 2)

        @pl.loop(0, n_chunks)
        def _(ci):
            off = base + ci * chunk
            t_base = (off // Bc) * Bc
            c = (off - t_base) // chunk
            for j in range(4):
                s = t_base + q4 * j + (chunk // 4) * c
                pltpu.sync_copy(idx_hbm.at[pl.ds(s, chunk // 4)],
                                src_v.at[pl.ds(j * (chunk // 4), chunk // 4)])

            @pl.loop(0, chunk // 16)
            def _(m):
                g = plsc.load_gather(src_v, [pat + 4 * m])
                rid = 4 * (g & mask) + (g >> shift)
                idx_v[pl.ds(m * 16, 16)] = rid

            pltpu.sync_copy(idx_v, out_hbm.at[pl.ds(off, chunk)])

    return permute_kernel(ids_tm)


def _sc_gather(table, ids_f, n):
    d = table.shape[1]
    b_per_w = n // _NW
    chunk = 1024
    n_chunks = b_per_w // chunk
    assert n_chunks % 2 == 1
    mesh = plsc.VectorSubcoreMesh(core_axis_name="c", subcore_axis_name="s")

    @functools.partial(
        pl.kernel,
        mesh=mesh,
        out_type=jax.ShapeDtypeStruct((n, d), table.dtype),
        compiler_params=pltpu.CompilerParams(
            use_tc_tiling_on_sc=False, needs_layout_passes=False),
        scratch_types=[
            pltpu.VMEM((chunk,), jnp.int32),
            pltpu.VMEM((chunk,), jnp.int32),
            pltpu.VMEM((chunk, d), table.dtype),
            pltpu.VMEM((chunk, d), table.dtype),
            pltpu.SemaphoreType.DMA,
            pltpu.SemaphoreType.DMA,
            pltpu.SemaphoreType.DMA,
            pltpu.SemaphoreType.DMA,
            pltpu.SemaphoreType.DMA,
            pltpu.SemaphoreType.DMA,
        ],
    )
    def gather_kernel(table_hbm, idx_hbm, out_hbm,
                      idx_v0, idx_v1, rows_v0, rows_v1,
                      si0, si1, sg0, sg1, so0, so1):
        wid = lax.axis_index("s") * _NC + lax.axis_index("c")
        base = wid * b_per_w
        idx_v = (idx_v0, idx_v1)
        rows_v = (rows_v0, rows_v1)
        si = (si0, si1)
        sg = (sg0, sg1)
        so = (so0, so1)

        def do_chunk(cc, b, prefetch, out_wait):
            off = base + cc * chunk
            if prefetch:
                @pl.when(cc + 1 < n_chunks)
                def _():
                    noff = base + (cc + 1) * chunk
                    pltpu.async_copy(idx_hbm.at[pl.ds(noff, chunk)],
                                     idx_v[1 - b], si[1 - b])
            pltpu.make_async_copy(idx_hbm.at[pl.ds(off, chunk)],
                                  idx_v[b], si[b]).wait()
            if out_wait:
                @pl.when(cc >= 2)
                def _():
                    pltpu.make_async_copy(rows_v[b],
                                          out_hbm.at[pl.ds(off, chunk)],
                                          so[b]).wait()
            pltpu.async_copy(table_hbm.at[idx_v[b]], rows_v[b], sg[b]).wait()
            pltpu.async_copy(rows_v[b], out_hbm.at[pl.ds(off, chunk)], so[b])

        pltpu.async_copy(idx_hbm.at[pl.ds(base, chunk)], idx_v0, si0)

        @pl.loop(0, n_chunks - 1, step=2)
        def _(ci):
            do_chunk(ci, 0, True, True)
            do_chunk(ci + 1, 1, True, True)

        do_chunk(n_chunks - 1, 0, False, True)

        pltpu.make_async_copy(
            rows_v1, out_hbm.at[pl.ds(base + (n_chunks - 2) * chunk, chunk)],
            so1).wait()
        pltpu.make_async_copy(
            rows_v0, out_hbm.at[pl.ds(base + (n_chunks - 1) * chunk, chunk)],
            so0).wait()

    return gather_kernel(table, ids_f)


def _tc_proj(x2, Wblk, bcol, Tc, Bc, D):
    q = Bc // 4
    tblk = 4

    def proj_body(x_ref, w_ref, b_ref, o_ref):
        dn = (((1,), (1,)), ((), ()))
        for tt in range(tblk):
            xt = x_ref[tt * q:(tt + 1) * q, :]
            z = lax.dot_general(w_ref[...], xt, dn,
                                preferred_element_type=jnp.float32)
            z = z + b_ref[...]
            for j in range(4):
                o_ref[tt, :, j * q:(j + 1) * q] = z[j * D:(j + 1) * D, :]

    return pl.pallas_call(
        proj_body,
        grid=(Tc // tblk,),
        in_specs=[
            pl.BlockSpec((tblk * q, 4 * D), lambda i: (i, 0)),
            pl.BlockSpec((4 * D, 4 * D), lambda i: (0, 0)),
            pl.BlockSpec((4 * D, 1), lambda i: (0, 0)),
        ],
        out_specs=pl.BlockSpec((tblk, D, Bc), lambda i: (i, 0, 0)),
        out_shape=jax.ShapeDtypeStruct((Tc, D, Bc), jnp.float32),
    )(x2, Wblk, bcol)


def kernel(input_ids, embed, W, b):
    Bc, Tc = input_ids.shape
    V, D = embed.shape
    n = Bc * Tc
    q = Bc // 4

    shift = max(int(V - 1).bit_length() - 2, 1)
    v4p = 1 << shift

    table4 = _tc_repack(embed.T, v4p)
    table_lin = table4.reshape(4 * v4p, D)

    ids_tm = input_ids.T.reshape(n)
    ids_f = _sc_permute_ids(ids_tm, n, Bc, shift)
    x = _sc_gather(table_lin, ids_f, n)
    x2 = x.reshape(n // 4, 4 * D)

    Wblk = jnp.kron(jnp.eye(4, dtype=W.dtype), W)
    bcol = jnp.tile(b, 4)[:, None]
    yt = _tc_proj(x2, Wblk, bcol, Tc, Bc, D)

    return jnp.transpose(yt, (2, 0, 1))


# speedup vs baseline: 1.0341x; 1.0186x over previous
import functools

import jax
import jax.numpy as jnp
from jax import lax
from jax.experimental import pallas as pl
from jax.experimental.pallas import tpu as pltpu
from jax.experimental.pallas import tpu_sc as plsc

_NC = 2
_NS = 16
_NW = _NC * _NS


def _tc_repack(embed_t, v4p):
    D, V = embed_t.shape
    cols = 8192
    nblk = v4p // cols
    last_blk = (V - 1) // cols

    def repack_body(x0, x1, x2, x3, o_ref):
        xcat = jnp.concatenate(
            [x0[...], x1[...], x2[...], x3[...]], axis=0)
        o_ref[...] = xcat.T

    in_specs = [
        pl.BlockSpec(
            (D, cols),
            functools.partial(
                lambda j, i: (0, jnp.minimum(i + j * nblk, last_blk)), j))
        for j in range(4)
    ]
    return pl.pallas_call(
        repack_body,
        grid=(nblk,),
        in_specs=in_specs,
        out_specs=pl.BlockSpec((cols, 4 * D), lambda i: (i, 0)),
        out_shape=jax.ShapeDtypeStruct((v4p, 4 * D), embed_t.dtype),
    )(embed_t, embed_t, embed_t, embed_t)


def _sc_permute_ids(ids_tm, n, Bc, shift):
    q4 = Bc // 4
    b_per_w = n // _NW
    chunk = 1024
    n_chunks = b_per_w // chunk
    mask = (1 << shift) - 1
    mesh = plsc.VectorSubcoreMesh(core_axis_name="c", subcore_axis_name="s")

    @functools.partial(
        pl.kernel,
        mesh=mesh,
        out_type=jax.ShapeDtypeStruct((n,), jnp.int32),
        compiler_params=pltpu.CompilerParams(
            use_tc_tiling_on_sc=False, needs_layout_passes=False),
        scratch_types=[
            pltpu.VMEM((chunk,), jnp.int32),
            pltpu.VMEM((chunk,), jnp.int32),
        ],
    )
    def permute_kernel(idx_hbm, out_hbm, src_v, idx_v):
        wid = lax.axis_index("s") * _NC + lax.axis_index("c")
        base = wid * b_per_w
        r = lax.iota(jnp.int32, 16)
        pat = (chunk // 4) * (r & 3) + (r >> 2)

        @pl.loop(0, n_chunks)
        def _(ci):
            off = base + ci * chunk
            t_base = (off // Bc) * Bc
            c = (off - t_base) // chunk
            for j in range(4):
                s = t_base + q4 * j + (chunk // 4) * c
                pltpu.sync_copy(idx_hbm.at[pl.ds(s, chunk // 4)],
                                src_v.at[pl.ds(j * (chunk // 4), chunk // 4)])

            @pl.loop(0, chunk // 16)
            def _(m):
                g = plsc.load_gather(src_v, [pat + 4 * m])
                rid = 4 * (g & mask) + (g >> shift)
                idx_v[pl.ds(m * 16, 16)] = rid

            pltpu.sync_copy(idx_v, out_hbm.at[pl.ds(off, chunk)])

    return permute_kernel(ids_tm)


def _sc_gather(table, ids_f, n):
    d = table.shape[1]
    b_per_w = n // _NW
    chunk = 1024
    n_chunks = b_per_w // chunk
    assert n_chunks % 2 == 1
    mesh = plsc.VectorSubcoreMesh(core_axis_name="c", subcore_axis_name="s")

    @functools.partial(
        pl.kernel,
        mesh=mesh,
        out_type=jax.ShapeDtypeStruct((n, d), table.dtype),
        compiler_params=pltpu.CompilerParams(
            use_tc_tiling_on_sc=False, needs_layout_passes=False),
        scratch_types=[
            pltpu.VMEM((chunk,), jnp.int32),
            pltpu.VMEM((chunk,), jnp.int32),
            pltpu.VMEM((chunk, d), table.dtype),
            pltpu.VMEM((chunk, d), table.dtype),
            pltpu.SemaphoreType.DMA,
            pltpu.SemaphoreType.DMA,
            pltpu.SemaphoreType.DMA,
            pltpu.SemaphoreType.DMA,
            pltpu.SemaphoreType.DMA,
            pltpu.SemaphoreType.DMA,
        ],
    )
    def gather_kernel(table_hbm, idx_hbm, out_hbm,
                      idx_v0, idx_v1, rows_v0, rows_v1,
                      si0, si1, sg0, sg1, so0, so1):
        wid = lax.axis_index("s") * _NC + lax.axis_index("c")
        base = wid * b_per_w
        idx_v = (idx_v0, idx_v1)
        rows_v = (rows_v0, rows_v1)
        si = (si0, si1)
        sg = (sg0, sg1)
        so = (so0, so1)

        def do_chunk(cc, b, prefetch, out_wait):
            off = base + cc * chunk
            if prefetch:
                @pl.when(cc + 1 < n_chunks)
                def _():
                    noff = base + (cc + 1) * chunk
                    pltpu.async_copy(idx_hbm.at[pl.ds(noff, chunk)],
                                     idx_v[1 - b], si[1 - b])
            pltpu.make_async_copy(idx_hbm.at[pl.ds(off, chunk)],
                                  idx_v[b], si[b]).wait()
            if out_wait:
                @pl.when(cc >= 2)
                def _():
                    pltpu.make_async_copy(rows_v[b],
                                          out_hbm.at[pl.ds(off, chunk)],
                                          so[b]).wait()
            pltpu.async_copy(table_hbm.at[idx_v[b]], rows_v[b], sg[b]).wait()
            pltpu.async_copy(rows_v[b], out_hbm.at[pl.ds(off, chunk)], so[b])

        pltpu.async_copy(idx_hbm.at[pl.ds(base, chunk)], idx_v0, si0)

        @pl.loop(0, n_chunks - 1, step=2)
        def _(ci):
            do_chunk(ci, 0, True, True)
            do_chunk(ci + 1, 1, True, True)

        do_chunk(n_chunks - 1, 0, False, True)

        pltpu.make_async_copy(
            rows_v1, out_hbm.at[pl.ds(base + (n_chunks - 2) * chunk, chunk)],
            so1).wait()
        pltpu.make_async_copy(
            rows_v0, out_hbm.at[pl.ds(base + (n_chunks - 1) * chunk, chunk)],
            so0).wait()

    return gather_kernel(table, ids_f)


def _tc_proj(x2, Wblk, bcol, Tc, Bc, D):
    q = Bc // 4
    tblk = 8

    def proj_body(x_ref, w_ref, b_ref, o_ref):
        dn = (((1,), (1,)), ((), ()))
        for tt in range(tblk):
            xt = x_ref[tt * q:(tt + 1) * q, :]
            z = lax.dot_general(w_ref[...], xt, dn,
                                preferred_element_type=jnp.float32)
            z = z + b_ref[...]
            for j in range(4):
                o_ref[tt, :, j * q:(j + 1) * q] = z[j * D:(j + 1) * D, :]

    return pl.pallas_call(
        proj_body,
        grid=(Tc // tblk,),
        in_specs=[
            pl.BlockSpec((tblk * q, 4 * D), lambda i: (i, 0)),
            pl.BlockSpec((4 * D, 4 * D), lambda i: (0, 0)),
            pl.BlockSpec((4 * D, 1), lambda i: (0, 0)),
        ],
        out_specs=pl.BlockSpec((tblk, D, Bc), lambda i: (i, 0, 0)),
        out_shape=jax.ShapeDtypeStruct((Tc, D, Bc), jnp.float32),
    )(x2, Wblk, bcol)


def kernel(input_ids, embed, W, b):
    Bc, Tc = input_ids.shape
    V, D = embed.shape
    n = Bc * Tc
    q = Bc // 4

    shift = max(int(V - 1).bit_length() - 2, 1)
    v4p = 1 << shift

    table4 = _tc_repack(embed.T, v4p)
    table_lin = table4.reshape(4 * v4p, D)

    ids_tm = input_ids.T.reshape(n)
    ids_f = _sc_permute_ids(ids_tm, n, Bc, shift)
    x = _sc_gather(table_lin, ids_f, n)
    x2 = x.reshape(n // 4, 4 * D)

    Wblk = jnp.kron(jnp.eye(4, dtype=W.dtype), W)
    bcol = jnp.tile(b, 4)[:, None]
    yt = _tc_proj(x2, Wblk, bcol, Tc, Bc, D)

    return jnp.transpose(yt, (2, 0, 1))
